# baseline (speedup 1.0000x reference)
import dataclasses

import jax
import jax.numpy as jnp
from jax import lax
from jax.experimental import pallas as pl
from jax.experimental.pallas import tpu as pltpu
from jax.experimental.pallas import tpu_sc as plsc

_N_CLASSES = 1000
_C_PAD = 1024
_N_EMB = 64
_ACC_W = 80
_BATCH = 16384
_NC = 2
_NS = 16
_NW = _NC * _NS
_ROWS = _BATCH // _NW
_XCHUNK = 128


def _sc_partials(x, y):
    zeros = jnp.zeros((_C_PAD * _ACC_W,), jnp.float32)
    mesh = plsc.VectorSubcoreMesh(core_axis_name="c", subcore_axis_name="s")
    cp = pltpu.CompilerParams()
    if "needs_layout_passes" in pltpu.CompilerParams.__dataclass_fields__:
        cp = dataclasses.replace(cp, needs_layout_passes=False)

    @pl.kernel(
        compiler_params=cp,
        out_type=jax.ShapeDtypeStruct((_NW, _C_PAD * _ACC_W), jnp.float32),
        mesh=mesh,
        scratch_types=[
            pltpu.VMEM((_XCHUNK, _N_EMB), jnp.float32),
            pltpu.VMEM((_ROWS,), jnp.int32),
            pltpu.VMEM((_C_PAD * _ACC_W,), jnp.float32),
        ],
    )
    def k(x_hbm, y_hbm, z_hbm, out_hbm, x_v, y_v, acc_v):
        cid = lax.axis_index("c")
        sid = lax.axis_index("s")
        wid = cid * _NS + sid
        base = wid * _ROWS

        pltpu.sync_copy(z_hbm, acc_v)
        pltpu.sync_copy(y_hbm.at[pl.ds(base, _ROWS)], y_v)

        iota = lax.iota(jnp.int32, 16)
        ones = jnp.ones((16,), jnp.float32)

        @pl.loop(0, _ROWS // _XCHUNK)
        def _chunk(c):
            pltpu.sync_copy(x_hbm.at[pl.ds(base + c * _XCHUNK, _XCHUNK)], x_v)

            @pl.loop(0, _XCHUNK // 16)
            def _group(g):
                for l in range(16):
                    r = g * 16 + l
                    sel = jnp.full((16,), 0, jnp.int32) + (c * _XCHUNK + r)
                    row_base = plsc.load_gather(y_v, [sel]) * _ACC_W
                    for j in range(4):
                        idx = row_base + (iota + j * 16)
                        val = x_v[r, pl.ds(j * 16, 16)]
                        plsc.addupdate_scatter(acc_v, [idx], val)
                    plsc.addupdate_scatter(acc_v, [row_base + (iota + 64)], ones)

        pltpu.sync_copy(acc_v, out_hbm.at[wid])

    return k(x, y, zeros)


def _tc_body(p_ref, centers_ref, nbt_ref, out_ref, acc_ref):
    i = pl.program_id(0)

    @pl.when(i == 0)
    def _init():
        acc_ref[...] = jnp.zeros_like(acc_ref)

    acc_ref[...] += p_ref[0]

    @pl.when(i == _NW - 1)
    def _fin():
        sums = acc_ref[0:_N_CLASSES, 0:_N_EMB]
        counts = acc_ref[0:_N_CLASSES, 64:65]
        present = counts > 0.0
        denom = jnp.where(present, counts, 1.0)
        mu = sums / denom
        nbt = nbt_ref[0, 0]
        centers = centers_ref[...]
        cma = (mu + centers * nbt) / (nbt + 1.0)
        out_ref[...] = jnp.where(present, cma, centers)


def _tc_finalize(partials, centers, nbt):
    p3 = partials.reshape(_NW, _C_PAD, _ACC_W)
    nbt2 = nbt.reshape(1, 1)
    return pl.pallas_call(
        _tc_body,
        grid=(_NW,),
        in_specs=[
            pl.BlockSpec((1, _C_PAD, _ACC_W), lambda i: (i, 0, 0)),
            pl.BlockSpec((_N_CLASSES, _N_EMB), lambda i: (0, 0)),
            pl.BlockSpec((1, 1), lambda i: (0, 0)),
        ],
        out_specs=pl.BlockSpec((_N_CLASSES, _N_EMB), lambda i: (0, 0)),
        out_shape=jax.ShapeDtypeStruct((_N_CLASSES, _N_EMB), jnp.float32),
        scratch_shapes=[pltpu.VMEM((_C_PAD, _ACC_W), jnp.float32)],
    )(p3, centers, nbt2)


def kernel(x, y, centers, num_batches_tracked):
    partials = _sc_partials(x, y)
    new_centers = _tc_finalize(partials, centers, num_batches_tracked)
    return (x, new_centers)
